# Initial kernel scaffold; baseline (speedup 1.0000x reference)
import jax
import jax.numpy as jnp
from jax.experimental import pallas as pl


def kernel(input_token_id, input_position_id, token_table, pos_table, ln_gamma, ln_beta):
    raise NotImplementedError("write your pallas kernel here")


# speedup vs baseline: 4.4667x; 4.4667x over previous
import functools

import jax
import jax.numpy as jnp
from jax import lax
from jax.experimental import pallas as pl
from jax.experimental.pallas import tpu as pltpu
from jax.experimental.pallas import tpu_sc as plsc

B = 4096
L = 200
DIM = 64
N = B * L

NC = 2
NS = 16
NW = NC * NS
PER_W = N // NW
CHUNK = 256
N_CHUNKS = PER_W // CHUNK

_mesh = plsc.VectorSubcoreMesh(core_axis_name="c", subcore_axis_name="s")


@functools.partial(
    pl.kernel,
    out_type=jax.ShapeDtypeStruct((N, DIM), jnp.float32),
    mesh=_mesh,
    compiler_params=pltpu.CompilerParams(use_tc_tiling_on_sc=False),
    scratch_types=[
        pltpu.VMEM((CHUNK,), jnp.int32),
        pltpu.VMEM((CHUNK,), jnp.int32),
        pltpu.VMEM((CHUNK, DIM), jnp.float32),
        pltpu.VMEM((CHUNK, DIM), jnp.float32),
        pltpu.VMEM((CHUNK, DIM), jnp.float32),
        pltpu.VMEM((DIM,), jnp.float32),
        pltpu.VMEM((DIM,), jnp.float32),
        pltpu.SemaphoreType.DMA,
        pltpu.SemaphoreType.DMA,
    ],
)
def _embed_ln_kernel(tok_hbm, pos_hbm, ttab_hbm, ptab_hbm, gamma_hbm, beta_hbm,
                     out_hbm,
                     idxt_v, idxp_v, trows_v, prows_v, orows_v,
                     gamma_v, beta_v, sem_t, sem_p):
    wid = lax.axis_index("s") * NC + lax.axis_index("c")
    base_w = wid * PER_W

    pltpu.sync_copy(gamma_hbm, gamma_v)
    pltpu.sync_copy(beta_hbm, beta_v)
    g = [gamma_v[pl.ds(k * 16, 16)] for k in range(DIM // 16)]
    bt = [beta_v[pl.ds(k * 16, 16)] for k in range(DIM // 16)]

    def chunk_body(ci, carry):
        base = base_w + ci * CHUNK
        pltpu.sync_copy(tok_hbm.at[pl.ds(base, CHUNK)], idxt_v)
        pltpu.sync_copy(pos_hbm.at[pl.ds(base, CHUNK)], idxp_v)
        cp_t = pltpu.async_copy(ttab_hbm.at[idxt_v], trows_v, sem_t)
        cp_p = pltpu.async_copy(ptab_hbm.at[idxp_v], prows_v, sem_p)
        cp_t.wait()
        cp_p.wait()

        lane = lax.iota(jnp.int32, 16)
        perms = [lane ^ sh for sh in (1, 2, 4, 8)]

        def allsum(v):
            for p in perms:
                v = v + v.at[p].get(mode="promise_in_bounds")
            return v

        def tok_body(t, tc):
            e = [trows_v[t, pl.ds(k * 16, 16)] + prows_v[t, pl.ds(k * 16, 16)]
                 for k in range(DIM // 16)]
            s = (e[0] + e[1]) + (e[2] + e[3])
            q = (e[0] * e[0] + e[1] * e[1]) + (e[2] * e[2] + e[3] * e[3])
            mean = allsum(s) * (1.0 / DIM)
            var = allsum(q) * (1.0 / DIM) - mean * mean
            xv = jnp.maximum(var, 0.0) + 1e-12
            iv = lax.bitcast_convert_type(xv, jnp.int32)
            iv = 0x5F3759DF - (iv >> 1)
            y = lax.bitcast_convert_type(iv, jnp.float32)
            hx = xv * 0.5
            for _ in range(3):
                y = y * (1.5 - hx * y * y)
            for k in range(DIM // 16):
                orows_v[t, pl.ds(k * 16, 16)] = (e[k] - mean) * y * g[k] + bt[k]
            return tc

        lax.fori_loop(0, CHUNK, tok_body, 0)
        pltpu.sync_copy(orows_v, out_hbm.at[pl.ds(base, CHUNK)])
        return carry

    lax.fori_loop(0, N_CHUNKS, chunk_body, 0)


def kernel(input_token_id, input_position_id, token_table, pos_table,
           ln_gamma, ln_beta):
    tok = jnp.asarray(input_token_id, jnp.int32).reshape(N)
    pos = jnp.asarray(input_position_id, jnp.int32).reshape(N)
    out = _embed_ln_kernel(tok, pos, token_table, pos_table, ln_gamma, ln_beta)
    return out.reshape(B, L, DIM)
